# baseline (device time: 12805 ns/iter reference)
import jax
import jax.numpy as jnp
from jax import lax
from jax.experimental import pallas as pl
from jax.experimental.pallas import tpu as pltpu

N_CHUNKS = 4


def kernel(A, B):
    m, k = A.shape
    _, n = B.shape
    ck = m // N_CHUNKS

    def body(a_hbm, b_hbm, out_hbm,
             a_v, b_v, b_bf, out_v, xsend_buf, xrecv_buf,
             load_sems, out_sems, xsend_sems, xrecv_sems):
        my_x = lax.axis_index("x")
        my_y = lax.axis_index("y")
        xpeer = (1 - my_x, my_y)

        barrier_sem = pltpu.get_barrier_semaphore()
        pl.semaphore_signal(
            barrier_sem, inc=1, device_id=xpeer,
            device_id_type=pl.DeviceIdType.MESH,
        )

        cp_a = pltpu.make_async_copy(a_hbm, a_v, load_sems.at[0])
        cp_b = pltpu.make_async_copy(b_hbm, b_v, load_sems.at[1])
        cp_a.start()
        cp_b.start()
        cp_b.wait()
        b_bf[...] = b_v[...].astype(jnp.bfloat16)
        cp_a.wait()

        def x_rdma(c):
            sl = pl.ds(c * ck, ck)
            return pltpu.make_async_remote_copy(
                src_ref=xsend_buf.at[sl],
                dst_ref=xrecv_buf.at[sl],
                send_sem=xsend_sems.at[c],
                recv_sem=xrecv_sems.at[c],
                device_id=xpeer,
                device_id_type=pl.DeviceIdType.MESH,
            )

        def out_dma(c):
            sl = pl.ds(c * ck, ck)
            return pltpu.make_async_copy(
                out_v.at[sl], out_hbm.at[sl], out_sems.at[c]
            )

        for c in range(N_CHUNKS):
            sl = pl.ds(c * ck, ck)
            xsend_buf[sl, :] = jnp.dot(
                a_v[sl, :].astype(jnp.bfloat16),
                b_bf[...],
                preferred_element_type=jnp.float32,
            ).astype(jnp.bfloat16)
            if c == 0:
                pl.semaphore_wait(barrier_sem, 1)
            x_rdma(c).start()

        for c in range(N_CHUNKS):
            x_rdma(c).wait_recv()
            sl = pl.ds(c * ck, ck)
            out_v[sl, :] = xsend_buf[sl, :] + xrecv_buf[sl, :]
            out_dma(c).start()

        for c in range(N_CHUNKS):
            out_dma(c).wait()
            x_rdma(c).wait_send()

    return pl.pallas_call(
        body,
        out_shape=jax.ShapeDtypeStruct((m, n), jnp.bfloat16),
        in_specs=[
            pl.BlockSpec(memory_space=pltpu.MemorySpace.HBM),
            pl.BlockSpec(memory_space=pltpu.MemorySpace.HBM),
        ],
        out_specs=pl.BlockSpec(memory_space=pltpu.MemorySpace.HBM),
        scratch_shapes=[
            pltpu.VMEM((m, k), jnp.float32),
            pltpu.VMEM((k, n), jnp.float32),
            pltpu.VMEM((k, n), jnp.bfloat16),
            pltpu.VMEM((m, n), jnp.bfloat16),
            pltpu.VMEM((m, n), jnp.bfloat16),
            pltpu.VMEM((m, n), jnp.bfloat16),
            pltpu.SemaphoreType.DMA((2,)),
            pltpu.SemaphoreType.DMA((N_CHUNKS,)),
            pltpu.SemaphoreType.DMA((N_CHUNKS,)),
            pltpu.SemaphoreType.DMA((N_CHUNKS,)),
        ],
        compiler_params=pltpu.CompilerParams(collective_id=0),
    )(A, B)


# device time: 12505 ns/iter; 1.0240x vs baseline; 1.0240x over previous
import jax
import jax.numpy as jnp
from jax import lax
from jax.experimental import pallas as pl
from jax.experimental.pallas import tpu as pltpu

N_CHUNKS = 4


def kernel(A, B):
    m, k = A.shape
    _, n = B.shape
    ck = m // N_CHUNKS

    def body(a_ref, b_ref, out_ref, xsend_buf, xrecv_buf,
             xsend_sems, xrecv_sems):
        my_x = lax.axis_index("x")
        my_y = lax.axis_index("y")
        xpeer = (1 - my_x, my_y)

        barrier_sem = pltpu.get_barrier_semaphore()
        pl.semaphore_signal(
            barrier_sem, inc=1, device_id=xpeer,
            device_id_type=pl.DeviceIdType.MESH,
        )

        def x_rdma(c):
            sl = pl.ds(c * ck, ck)
            return pltpu.make_async_remote_copy(
                src_ref=xsend_buf.at[sl],
                dst_ref=xrecv_buf.at[sl],
                send_sem=xsend_sems.at[c],
                recv_sem=xrecv_sems.at[c],
                device_id=xpeer,
                device_id_type=pl.DeviceIdType.MESH,
            )

        for c in range(N_CHUNKS):
            sl = pl.ds(c * ck, ck)
            xsend_buf[sl, :] = jnp.dot(
                a_ref[sl, :].astype(jnp.bfloat16),
                b_ref[...].astype(jnp.bfloat16),
                preferred_element_type=jnp.float32,
            ).astype(jnp.bfloat16)
            if c == 0:
                pl.semaphore_wait(barrier_sem, 1)
            x_rdma(c).start()

        for c in range(N_CHUNKS):
            x_rdma(c).wait_recv()
            sl = pl.ds(c * ck, ck)
            out_ref[sl, :] = xsend_buf[sl, :] + xrecv_buf[sl, :]

        for c in range(N_CHUNKS):
            x_rdma(c).wait_send()

    return pl.pallas_call(
        body,
        out_shape=jax.ShapeDtypeStruct((m, n), jnp.bfloat16),
        in_specs=[
            pl.BlockSpec(memory_space=pltpu.VMEM),
            pl.BlockSpec(memory_space=pltpu.VMEM),
        ],
        out_specs=pl.BlockSpec(memory_space=pltpu.VMEM),
        scratch_shapes=[
            pltpu.VMEM((m, n), jnp.bfloat16),
            pltpu.VMEM((m, n), jnp.bfloat16),
            pltpu.SemaphoreType.DMA((N_CHUNKS,)),
            pltpu.SemaphoreType.DMA((N_CHUNKS,)),
        ],
        compiler_params=pltpu.CompilerParams(collective_id=0),
    )(A, B)
